# baseline (speedup 1.0000x reference)
import jax
import jax.numpy as jnp
from jax.experimental import pallas as pl
from jax.experimental.pallas import tpu as pltpu

B, S, D = 64, 2048, 256
S_CHUNK = 256
N_CHUNKS = S // S_CHUNK


def _attn_kernel(x_ref, wt_ref, b_ref, ctx_ref, o_ref, d_ref, acc_ref):
    s = pl.program_id(1)

    @pl.when(s == 0)
    def _():
        d_ref[...] = jnp.zeros_like(d_ref)
        acc_ref[...] = jnp.zeros_like(acc_ref)

    x = x_ref[0]
    u = jnp.tanh(
        jnp.dot(x, wt_ref[...], preferred_element_type=jnp.float32) + b_ref[...]
    )
    scores = jax.lax.dot_general(
        ctx_ref[...], u, (((1,), (1,)), ((), ())),
        preferred_element_type=jnp.float32,
    )
    p = jnp.exp(scores)
    d_ref[...] += jnp.sum(p, axis=1, keepdims=True)
    acc_ref[...] += jnp.dot(p, x, preferred_element_type=jnp.float32)

    @pl.when(s == N_CHUNKS - 1)
    def _():
        o_ref[...] = (acc_ref[...] / d_ref[...])[None]


def kernel(lstm_output, W_attn, b_attn, ctx):
    wt = W_attn.T
    b2 = b_attn[None, :]
    ctx2 = ctx[None, :]
    out = pl.pallas_call(
        _attn_kernel,
        grid=(B, N_CHUNKS),
        in_specs=[
            pl.BlockSpec((1, S_CHUNK, D), lambda b, s: (b, s, 0)),
            pl.BlockSpec((D, D), lambda b, s: (0, 0)),
            pl.BlockSpec((1, D), lambda b, s: (0, 0)),
            pl.BlockSpec((1, D), lambda b, s: (0, 0)),
        ],
        out_specs=pl.BlockSpec((1, 1, D), lambda b, s: (b, 0, 0)),
        out_shape=jax.ShapeDtypeStruct((B, 1, D), jnp.float32),
        scratch_shapes=[
            pltpu.VMEM((1, 1), jnp.float32),
            pltpu.VMEM((1, D), jnp.float32),
        ],
        compiler_params=pltpu.CompilerParams(
            dimension_semantics=("parallel", "arbitrary"),
        ),
    )(lstm_output, wt, b2, ctx2)
    return out.reshape(1, B, D)


# speedup vs baseline: 4.2044x; 4.2044x over previous
import jax
import jax.numpy as jnp
from jax.experimental import pallas as pl
from jax.experimental.pallas import tpu as pltpu

B, S, D = 64, 2048, 256
S_CHUNK = 2048
N_CHUNKS = S // S_CHUNK


def _attn_kernel(x_ref, wt_ref, b_ref, ctx_ref, o_ref, d_ref, acc_ref):
    s = pl.program_id(1)

    @pl.when(s == 0)
    def _():
        d_ref[...] = jnp.zeros_like(d_ref)
        acc_ref[...] = jnp.zeros_like(acc_ref)

    x = x_ref[0]
    u = jnp.tanh(
        jnp.dot(x, wt_ref[...], preferred_element_type=jnp.float32) + b_ref[...]
    )
    scores = jax.lax.dot_general(
        ctx_ref[...], u, (((1,), (1,)), ((), ())),
        preferred_element_type=jnp.float32,
    )
    p = jnp.exp(scores)
    d_ref[...] += jnp.sum(p, axis=1, keepdims=True)
    acc_ref[...] += jnp.dot(p, x, preferred_element_type=jnp.float32)

    @pl.when(s == N_CHUNKS - 1)
    def _():
        o_ref[...] = (acc_ref[...] / d_ref[...])[None]


def kernel(lstm_output, W_attn, b_attn, ctx):
    wt = W_attn.T
    b2 = b_attn[None, :]
    ctx2 = ctx[None, :]
    out = pl.pallas_call(
        _attn_kernel,
        grid=(B, N_CHUNKS),
        in_specs=[
            pl.BlockSpec((1, S_CHUNK, D), lambda b, s: (b, s, 0)),
            pl.BlockSpec((D, D), lambda b, s: (0, 0)),
            pl.BlockSpec((1, D), lambda b, s: (0, 0)),
            pl.BlockSpec((1, D), lambda b, s: (0, 0)),
        ],
        out_specs=pl.BlockSpec((1, 1, D), lambda b, s: (b, 0, 0)),
        out_shape=jax.ShapeDtypeStruct((B, 1, D), jnp.float32),
        scratch_shapes=[
            pltpu.VMEM((1, 1), jnp.float32),
            pltpu.VMEM((1, D), jnp.float32),
        ],
        compiler_params=pltpu.CompilerParams(
            dimension_semantics=("parallel", "arbitrary"),
        ),
    )(lstm_output, wt, b2, ctx2)
    return out.reshape(1, B, D)


# speedup vs baseline: 4.2748x; 1.0167x over previous
import jax
import jax.numpy as jnp
from jax.experimental import pallas as pl
from jax.experimental.pallas import tpu as pltpu

B, S, D = 64, 2048, 256


def _attn_kernel(x_ref, wt_ref, b_ref, ctx_ref, o_ref):
    x = x_ref[0]
    u = jnp.tanh(
        jnp.dot(x, wt_ref[...], preferred_element_type=jnp.float32) + b_ref[...]
    )
    scores = jax.lax.dot_general(
        ctx_ref[...], u, (((1,), (1,)), ((), ())),
        preferred_element_type=jnp.float32,
    )
    p = jnp.exp(scores)
    d = jnp.sum(p, axis=1, keepdims=True)
    acc = jnp.dot(p, x, preferred_element_type=jnp.float32)
    o_ref[...] = (acc / d)[None]


def kernel(lstm_output, W_attn, b_attn, ctx):
    wt = W_attn.T
    b2 = b_attn[None, :]
    ctx2 = ctx[None, :]
    out = pl.pallas_call(
        _attn_kernel,
        grid=(B,),
        in_specs=[
            pl.BlockSpec((1, S, D), lambda b: (b, 0, 0)),
            pl.BlockSpec((D, D), lambda b: (0, 0)),
            pl.BlockSpec((1, D), lambda b: (0, 0)),
            pl.BlockSpec((1, D), lambda b: (0, 0)),
        ],
        out_specs=pl.BlockSpec((1, 1, D), lambda b: (b, 0, 0)),
        out_shape=jax.ShapeDtypeStruct((B, 1, D), jnp.float32),
        compiler_params=pltpu.CompilerParams(
            dimension_semantics=("parallel",),
        ),
    )(lstm_output, wt, b2, ctx2)
    return out.reshape(1, B, D)
